# Initial kernel scaffold; baseline (speedup 1.0000x reference)
import functools

import jax
import jax.numpy as jnp
from jax import lax
from jax.experimental import pallas as pl
from jax.experimental.pallas import tpu as pltpu
from jax.experimental.pallas import tpu_sc as plsc

VOCAB = 1000
NUM_ROWS = 1024 * 50
NUM_WORKERS = 32
ROWS_PER_WORKER = NUM_ROWS // NUM_WORKERS
CHUNK = 64
STEPS = ROWS_PER_WORKER // CHUNK

_mesh = plsc.VectorSubcoreMesh(core_axis_name="c", subcore_axis_name="s")


@functools.partial(
    pl.kernel,
    mesh=_mesh,
    out_type=jax.ShapeDtypeStruct((NUM_ROWS, VOCAB), jnp.float32),
    scratch_types=[
        pltpu.VMEM((CHUNK,), jnp.int32),
        pltpu.VMEM((CHUNK, VOCAB), jnp.float32),
        pltpu.SemaphoreType.DMA,
    ],
)
def _gather_rows(table_hbm, idx_hbm, out_hbm, idx_v, rows_v, sem):
    wid = lax.axis_index("s") * 2 + lax.axis_index("c")
    base = wid * ROWS_PER_WORKER

    @pl.loop(0, STEPS)
    def _(i):
        off = base + i * CHUNK
        pltpu.sync_copy(idx_hbm.at[pl.ds(off, CHUNK)], idx_v)
        pltpu.async_copy(table_hbm.at[idx_v], rows_v, sem).wait()
        pltpu.sync_copy(rows_v, out_hbm.at[pl.ds(off, CHUNK)])


def kernel(x, embedding_weight):
    idx = x.reshape(-1).astype(jnp.int32)
    out = _gather_rows(embedding_weight, idx)
    return out.reshape(x.shape[0], x.shape[1], VOCAB)


# speedup vs baseline: 1.0080x; 1.0080x over previous
import functools

import jax
import jax.numpy as jnp
from jax import lax
from jax.experimental import pallas as pl
from jax.experimental.pallas import tpu as pltpu
from jax.experimental.pallas import tpu_sc as plsc

VOCAB = 1000
NUM_ROWS = 1024 * 50
NUM_WORKERS = 32
ROWS_PER_WORKER = NUM_ROWS // NUM_WORKERS
CHUNK = 64
STEPS = ROWS_PER_WORKER // CHUNK

_mesh = plsc.VectorSubcoreMesh(core_axis_name="c", subcore_axis_name="s")


@functools.partial(
    pl.kernel,
    mesh=_mesh,
    out_type=jax.ShapeDtypeStruct((NUM_ROWS, VOCAB), jnp.float32),
    scratch_types=[
        pltpu.VMEM((CHUNK,), jnp.int32),
        pltpu.VMEM((CHUNK, VOCAB), jnp.float32),
        pltpu.SemaphoreType.DMA,
    ],
    compiler_params=pltpu.CompilerParams(use_tc_tiling_on_sc=False),
)
def _gather_rows(table_hbm, idx_hbm, out_hbm, idx_v, rows_v, sem):
    wid = lax.axis_index("s") * 2 + lax.axis_index("c")
    base = wid * ROWS_PER_WORKER

    @pl.loop(0, STEPS)
    def _(i):
        off = base + i * CHUNK
        pltpu.sync_copy(idx_hbm.at[pl.ds(off, CHUNK)], idx_v)
        pltpu.async_copy(table_hbm.at[idx_v], rows_v, sem).wait()
        pltpu.sync_copy(rows_v, out_hbm.at[pl.ds(off, CHUNK)])


def kernel(x, embedding_weight):
    idx = x.reshape(-1).astype(jnp.int32)
    out = _gather_rows(embedding_weight, idx)
    return out.reshape(x.shape[0], x.shape[1], VOCAB)


# speedup vs baseline: 1.0283x; 1.0201x over previous
import functools

import jax
import jax.numpy as jnp
from jax import lax
from jax.experimental import pallas as pl
from jax.experimental.pallas import tpu as pltpu
from jax.experimental.pallas import tpu_sc as plsc

VOCAB = 1000
NUM_ROWS = 1024 * 50
NUM_WORKERS = 32
ROWS_PER_WORKER = NUM_ROWS // NUM_WORKERS
CHUNK = 40
STEPS = ROWS_PER_WORKER // CHUNK

_mesh = plsc.VectorSubcoreMesh(core_axis_name="c", subcore_axis_name="s")


@functools.partial(
    pl.kernel,
    mesh=_mesh,
    out_type=jax.ShapeDtypeStruct((NUM_ROWS, VOCAB), jnp.float32),
    scratch_types=[
        pltpu.VMEM((ROWS_PER_WORKER,), jnp.int32),
        pltpu.VMEM((CHUNK, VOCAB), jnp.float32),
        pltpu.VMEM((CHUNK, VOCAB), jnp.float32),
        pltpu.SemaphoreType.DMA,
        pltpu.SemaphoreType.DMA,
    ],
    compiler_params=pltpu.CompilerParams(use_tc_tiling_on_sc=False),
)
def _gather_rows(table_hbm, idx_hbm, out_hbm, idx_v, rows0, rows1, sem0, sem1):
    wid = lax.axis_index("s") * 2 + lax.axis_index("c")
    base = wid * ROWS_PER_WORKER

    pltpu.sync_copy(idx_hbm.at[pl.ds(base, ROWS_PER_WORKER)], idx_v)

    def gather(i, rows_v, sem):
        return pltpu.async_copy(
            table_hbm.at[idx_v.at[pl.ds(i * CHUNK, CHUNK)]], rows_v, sem)

    def gather_wait(i, rows_v, sem):
        pltpu.make_async_copy(
            table_hbm.at[idx_v.at[pl.ds(i * CHUNK, CHUNK)]], rows_v, sem).wait()

    def put(i, rows_v):
        pltpu.sync_copy(rows_v, out_hbm.at[pl.ds(base + i * CHUNK, CHUNK)])

    gather(0, rows0, sem0)

    @pl.loop(0, STEPS // 2)
    def _(j):
        i0 = j * 2
        gather_wait(i0, rows0, sem0)
        gather(i0 + 1, rows1, sem1)
        put(i0, rows0)
        gather_wait(i0 + 1, rows1, sem1)

        @pl.when(j < STEPS // 2 - 1)
        def _():
            gather(i0 + 2, rows0, sem0)

        put(i0 + 1, rows1)


def kernel(x, embedding_weight):
    idx = x.reshape(-1).astype(jnp.int32)
    out = _gather_rows(embedding_weight, idx)
    return out.reshape(x.shape[0], x.shape[1], VOCAB)
